# baseline (device time: 35500 ns/iter reference)
import os

import jax
import jax.numpy as jnp
from jax import lax
from jax.experimental import pallas as pl
from jax.experimental.pallas import tpu as pltpu

N_DEV = 8
_VARIANT = os.environ.get("KVARIANT", "full")
_SCOPES = os.environ.get("KSCOPES", "0") == "1"

import contextlib


def _scope(name):
    return jax.named_scope(name) if _SCOPES else contextlib.nullcontext()


def _gelu(y):
    c = 0.7978845608028654
    return 0.5 * y * (1.0 + jnp.tanh(c * (y + 0.044715 * y * y * y)))


def kernel(x, w_mat):
    m_per, k = x.shape
    _, n = w_mat.shape
    n_per = n // N_DEV
    use_comm = _VARIANT != "nocomm"

    def body(x_hbm, w_hbm, out_ref, x_vmem, w_buf, send_buf, recv_buf,
             copy_sems, send_sems, recv_sems):
        me = lax.axis_index("i")

        x_copy = pltpu.make_async_copy(x_hbm, x_vmem, copy_sems.at[2])
        x_copy.start()

        def w_copy(d, slot):
            t = (me + d) % N_DEV
            return pltpu.make_async_copy(
                w_hbm.at[:, pl.ds(t * n_per, n_per)],
                w_buf.at[slot],
                copy_sems.at[slot],
            )

        w_copy(1, 0).start()

        if use_comm:
            with _scope("barrier"):
                barrier_sem = pltpu.get_barrier_semaphore()
                for d in range(1, N_DEV):
                    pl.semaphore_signal(
                        barrier_sem, inc=1,
                        device_id=((me + d) % N_DEV,),
                        device_id_type=pl.DeviceIdType.MESH,
                    )
                pl.semaphore_wait(barrier_sem, N_DEV - 1)

        with _scope("wait_x"):
            x_copy.wait()
            x_bf = x_vmem[...].astype(jnp.bfloat16)

        rdmas = []
        for d in range(1, N_DEV + 1):
            slot = (d - 1) % 2
            if d < N_DEV:
                w_copy(d + 1, 1 - slot).start()
            with _scope(f"wait_w{d}"):
                w_copy(d, slot).wait()
            with _scope(f"conv_w{d}"):
                w_blk = w_buf[slot].astype(jnp.bfloat16)
            if d == N_DEV:
                with _scope("dot_own"):
                    y_own = jnp.dot(
                        x_bf, w_blk, preferred_element_type=jnp.float32
                    )
                with _scope("gelu_own"):
                    out_ref[pl.ds(me * m_per, m_per), :] = _gelu(y_own)
                break
            with _scope(f"dot{d}"):
                send_buf[d - 1, :, :] = jnp.dot(
                    x_bf, w_blk, preferred_element_type=jnp.float32
                ).astype(jnp.bfloat16)
            if use_comm:
                t = (me + d) % N_DEV
                rdma = pltpu.make_async_remote_copy(
                    src_ref=send_buf.at[d - 1],
                    dst_ref=recv_buf.at[d - 1],
                    send_sem=send_sems.at[d - 1],
                    recv_sem=recv_sems.at[d - 1],
                    device_id=(t,),
                    device_id_type=pl.DeviceIdType.MESH,
                )
                rdma.start()
                rdmas.append(rdma)

        for d in range(1, N_DEV):
            if use_comm:
                with _scope(f"wait_recv{d}"):
                    rdmas[d - 1].wait_recv()
            s = (me - d) % N_DEV
            src = recv_buf if use_comm else send_buf
            with _scope(f"gelu{d}"):
                y = src[d - 1, :, :].astype(jnp.float32)
                out_ref[pl.ds(s * m_per, m_per), :] = _gelu(y)

        if use_comm:
            with _scope("wait_send"):
                for d in range(1, N_DEV):
                    rdmas[d - 1].wait_send()

    return pl.pallas_call(
        body,
        out_shape=jax.ShapeDtypeStruct((N_DEV * m_per, n_per), jnp.float32),
        in_specs=[
            pl.BlockSpec(memory_space=pl.ANY),
            pl.BlockSpec(memory_space=pl.ANY),
        ],
        out_specs=pl.BlockSpec(memory_space=pltpu.VMEM),
        scratch_shapes=[
            pltpu.VMEM((m_per, k), jnp.float32),
            pltpu.VMEM((2, k, n_per), jnp.float32),
            pltpu.VMEM((N_DEV - 1, m_per, n_per), jnp.bfloat16),
            pltpu.VMEM((N_DEV - 1, m_per, n_per), jnp.bfloat16),
            pltpu.SemaphoreType.DMA((3,)),
            pltpu.SemaphoreType.DMA((N_DEV - 1,)),
            pltpu.SemaphoreType.DMA((N_DEV - 1,)),
        ],
        compiler_params=pltpu.CompilerParams(
            collective_id=0 if use_comm else None,
            vmem_limit_bytes=100 * 1024 * 1024,
        ),
    )(x, w_mat)


# device time: 34475 ns/iter; 1.0297x vs baseline; 1.0297x over previous
import os

import jax
import jax.numpy as jnp
from jax import lax
from jax.experimental import pallas as pl
from jax.experimental.pallas import tpu as pltpu

N_DEV = 8
_VARIANT = os.environ.get("KVARIANT", "full")
_SCOPES = os.environ.get("KSCOPES", "0") == "1"

import contextlib


def _scope(name):
    return jax.named_scope(name) if _SCOPES else contextlib.nullcontext()


def _gelu(y):
    c = 0.7978845608028654
    return 0.5 * y * (1.0 + jnp.tanh(c * (y + 0.044715 * y * y * y)))


def kernel(x, w_mat):
    m_per, k = x.shape
    _, n = w_mat.shape
    n_per = n // N_DEV
    use_comm = _VARIANT != "nocomm"

    def body(x_hbm, w_hbm, out_hbm, x_vmem, w_buf, send_buf, recv_buf,
             out_stage, copy_sems, send_sems, recv_sems, out_sems):
        me = lax.axis_index("i")

        out_copies = []

        def store_out(i, s, block):
            out_stage[i, :, :] = block
            cp = pltpu.make_async_copy(
                out_stage.at[i],
                out_hbm.at[pl.ds(s * m_per, m_per), :],
                out_sems.at[i],
            )
            cp.start()
            out_copies.append(cp)

        x_copy = pltpu.make_async_copy(x_hbm, x_vmem, copy_sems.at[2])
        x_copy.start()

        def w_copy(d, slot):
            t = (me + d) % N_DEV
            return pltpu.make_async_copy(
                w_hbm.at[:, pl.ds(t * n_per, n_per)],
                w_buf.at[slot],
                copy_sems.at[slot],
            )

        w_copy(1, 0).start()

        if use_comm:
            with _scope("barrier"):
                barrier_sem = pltpu.get_barrier_semaphore()
                for d in range(1, N_DEV):
                    pl.semaphore_signal(
                        barrier_sem, inc=1,
                        device_id=((me + d) % N_DEV,),
                        device_id_type=pl.DeviceIdType.MESH,
                    )
                pl.semaphore_wait(barrier_sem, N_DEV - 1)

        with _scope("wait_x"):
            x_copy.wait()
            x_f32 = x_vmem[...]

        rdmas = []
        for d in range(1, N_DEV + 1):
            slot = (d - 1) % 2
            if d < N_DEV:
                w_copy(d + 1, 1 - slot).start()
            with _scope(f"wait_w{d}"):
                w_copy(d, slot).wait()
            w_blk = w_buf[slot]
            if d == N_DEV:
                with _scope("dot_own"):
                    y_own = jnp.dot(
                        x_f32, w_blk,
                        precision=lax.Precision.DEFAULT,
                        preferred_element_type=jnp.float32,
                    )
                with _scope("gelu_own"):
                    store_out(N_DEV - 1, me, _gelu(y_own))
                break
            with _scope(f"dot{d}"):
                send_buf[d - 1, :, :] = jnp.dot(
                    x_f32, w_blk,
                    precision=lax.Precision.DEFAULT,
                    preferred_element_type=jnp.float32,
                ).astype(jnp.bfloat16)
            if use_comm:
                t = (me + d) % N_DEV
                rdma = pltpu.make_async_remote_copy(
                    src_ref=send_buf.at[d - 1],
                    dst_ref=recv_buf.at[d - 1],
                    send_sem=send_sems.at[d - 1],
                    recv_sem=recv_sems.at[d - 1],
                    device_id=(t,),
                    device_id_type=pl.DeviceIdType.MESH,
                )
                rdma.start()
                rdmas.append(rdma)

        for d in range(1, N_DEV):
            if use_comm:
                with _scope(f"wait_recv{d}"):
                    rdmas[d - 1].wait_recv()
            s = (me - d) % N_DEV
            src = recv_buf if use_comm else send_buf
            with _scope(f"gelu{d}"):
                y = src[d - 1, :, :].astype(jnp.float32)
                store_out(d - 1, s, _gelu(y))

        with _scope("wait_out"):
            for cp in out_copies:
                cp.wait()

        if use_comm:
            with _scope("wait_send"):
                for d in range(1, N_DEV):
                    rdmas[d - 1].wait_send()

    return pl.pallas_call(
        body,
        out_shape=jax.ShapeDtypeStruct((N_DEV * m_per, n_per), jnp.float32),
        in_specs=[
            pl.BlockSpec(memory_space=pl.ANY),
            pl.BlockSpec(memory_space=pl.ANY),
        ],
        out_specs=pl.BlockSpec(memory_space=pl.ANY),
        scratch_shapes=[
            pltpu.VMEM((m_per, k), jnp.float32),
            pltpu.VMEM((2, k, n_per), jnp.float32),
            pltpu.VMEM((N_DEV - 1, m_per, n_per), jnp.bfloat16),
            pltpu.VMEM((N_DEV - 1, m_per, n_per), jnp.bfloat16),
            pltpu.VMEM((N_DEV, m_per, n_per), jnp.float32),
            pltpu.SemaphoreType.DMA((3,)),
            pltpu.SemaphoreType.DMA((N_DEV - 1,)),
            pltpu.SemaphoreType.DMA((N_DEV - 1,)),
            pltpu.SemaphoreType.DMA((N_DEV,)),
        ],
        compiler_params=pltpu.CompilerParams(
            collective_id=0 if use_comm else None,
            vmem_limit_bytes=100 * 1024 * 1024,
        ),
    )(x, w_mat)


# device time: 34432 ns/iter; 1.0310x vs baseline; 1.0012x over previous
import contextlib
import os

import jax
import jax.numpy as jnp
from jax import lax
from jax.experimental import pallas as pl
from jax.experimental.pallas import tpu as pltpu

N_DEV = 8
_VARIANT = os.environ.get("KVARIANT", "full")
_OUT_DTYPE = jnp.bfloat16 if os.environ.get("KOUT", "f32") == "bf16" else jnp.float32
_SCOPES = os.environ.get("KSCOPES", "0") == "1"


def _scope(name):
    return jax.named_scope(name) if _SCOPES else contextlib.nullcontext()


def _gelu(y):
    c = 0.7978845608028654
    return 0.5 * y * (1.0 + jnp.tanh(c * (y + 0.044715 * y * y * y)))


def kernel(x, w_mat):
    m_per, k = x.shape
    _, n = w_mat.shape
    n_per = n // N_DEV
    use_comm = _VARIANT != "nocomm"

    def body(x_hbm, w_hbm, out_hbm, x_vmem, w_buf, send_buf, recv_buf,
             out_stage, copy_sems, send_sems, recv_sems, out_sems):
        me = lax.axis_index("i")

        out_copies = []

        def store_out(i, s, block):
            out_stage[i, :, :] = block.astype(_OUT_DTYPE)
            cp = pltpu.make_async_copy(
                out_stage.at[i],
                out_hbm.at[pl.ds(s * m_per, m_per), :],
                out_sems.at[i],
            )
            cp.start()
            out_copies.append(cp)

        x_copy = pltpu.make_async_copy(x_hbm, x_vmem, copy_sems.at[2])
        x_copy.start()

        def w_copy(d, slot):
            t = (me + d) % N_DEV
            return pltpu.make_async_copy(
                w_hbm.at[:, pl.ds(t * n_per, n_per)],
                w_buf.at[slot],
                copy_sems.at[slot],
            )

        w_copy(1, 0).start()

        if use_comm:
            with _scope("barrier"):
                barrier_sem = pltpu.get_barrier_semaphore()
                for d in range(1, N_DEV):
                    pl.semaphore_signal(
                        barrier_sem, inc=1,
                        device_id=((me + d) % N_DEV,),
                        device_id_type=pl.DeviceIdType.MESH,
                    )
                pl.semaphore_wait(barrier_sem, N_DEV - 1)

        with _scope("wait_x"):
            x_copy.wait()
            x_f32 = x_vmem[...]

        def bdot(xv, wv):
            return jnp.dot(
                xv, wv,
                precision=lax.Precision.DEFAULT,
                preferred_element_type=jnp.float32,
            )

        rdmas = []
        for d in range(1, N_DEV + 1):
            slot = (d - 1) % 2
            if d < N_DEV:
                w_copy(d + 1, 1 - slot).start()
            with _scope(f"wait_w{d}"):
                w_copy(d, slot).wait()
            w_blk = w_buf[slot]
            if d == N_DEV:
                with _scope("dot_own"):
                    y_own = bdot(x_f32, w_blk)
                with _scope("gelu_own"):
                    store_out(N_DEV - 1, me, _gelu(y_own))
                break
            else:
                with _scope(f"dot{d}"):
                    send_buf[d - 1, :, :] = bdot(
                        x_f32, w_blk
                    ).astype(jnp.bfloat16)
            if use_comm:
                t = (me + d) % N_DEV
                rdma = pltpu.make_async_remote_copy(
                    src_ref=send_buf.at[d - 1],
                    dst_ref=recv_buf.at[d - 1],
                    send_sem=send_sems.at[d - 1],
                    recv_sem=recv_sems.at[d - 1],
                    device_id=(t,),
                    device_id_type=pl.DeviceIdType.MESH,
                )
                rdma.start()
                rdmas.append(rdma)

        for d in range(1, N_DEV):
            if use_comm:
                with _scope(f"wait_recv{d}"):
                    rdmas[d - 1].wait_recv()
            s = (me - d) % N_DEV
            src = recv_buf if use_comm else send_buf
            with _scope(f"gelu{d}"):
                y = src[d - 1, :, :].astype(jnp.float32)
                store_out(d - 1, s, _gelu(y))

        with _scope("wait_out"):
            for cp in out_copies:
                cp.wait()

        if use_comm:
            with _scope("wait_send"):
                for d in range(1, N_DEV):
                    rdmas[d - 1].wait_send()

    return pl.pallas_call(
        body,
        out_shape=jax.ShapeDtypeStruct((N_DEV * m_per, n_per), _OUT_DTYPE),
        in_specs=[
            pl.BlockSpec(memory_space=pl.ANY),
            pl.BlockSpec(memory_space=pl.ANY),
        ],
        out_specs=pl.BlockSpec(memory_space=pl.ANY),
        scratch_shapes=[
            pltpu.VMEM((m_per, k), jnp.float32),
            pltpu.VMEM((2, k, n_per), jnp.float32),
            pltpu.VMEM((N_DEV - 1, m_per, n_per), jnp.bfloat16),
            pltpu.VMEM((N_DEV - 1, m_per, n_per), jnp.bfloat16),
            pltpu.VMEM((N_DEV, m_per, n_per), _OUT_DTYPE),
            pltpu.SemaphoreType.DMA((4,)),
            pltpu.SemaphoreType.DMA((N_DEV - 1,)),
            pltpu.SemaphoreType.DMA((N_DEV - 1,)),
            pltpu.SemaphoreType.DMA((N_DEV,)),
        ],
        compiler_params=pltpu.CompilerParams(
            collective_id=0 if use_comm else None,
            vmem_limit_bytes=100 * 1024 * 1024,
        ),
    )(x, w_mat)
